# Initial kernel scaffold; baseline (speedup 1.0000x reference)
import functools

import jax
import jax.numpy as jnp
from jax import lax
from jax.experimental import pallas as pl
from jax.experimental.pallas import tpu as pltpu
from jax.experimental.pallas import tpu_sc as plsc

N, E, D, RBF, NG, L, NUM_ELEM = 10000, 320000, 128, 10, 64, 4, 84
NP = 10240
EP = 327680
CH = 128
NCHUNK = 80
BN = 1024
BE = 2048
ROWS_PER_SUB = NP // 16

_mesh = plsc.VectorSubcoreMesh(core_axis_name="c", subcore_axis_name="s")


def _embed_body(x_ref, emb_ref, wm0_ref, h_ref, hw_ref):
    xb = x_ref[0, :].reshape(BN, 1)
    lane = lax.broadcasted_iota(jnp.int32, (BN, D), 1)
    oh = (lane == xb).astype(jnp.float32)
    h = jnp.dot(oh, emb_ref[...], preferred_element_type=jnp.float32)
    h_ref[...] = h
    hw_ref[...] = jnp.dot(h, wm0_ref[...], preferred_element_type=jnp.float32)


def _embed(x2, emb_pad, wm0):
    return pl.pallas_call(
        _embed_body,
        grid=(NP // BN,),
        in_specs=[
            pl.BlockSpec((1, BN), lambda j: (j, 0)),
            pl.BlockSpec((D, D), lambda j: (0, 0)),
            pl.BlockSpec((D, D), lambda j: (0, 0)),
        ],
        out_specs=[
            pl.BlockSpec((BN, D), lambda j: (j, 0)),
            pl.BlockSpec((BN, D), lambda j: (j, 0)),
        ],
        out_shape=[
            jax.ShapeDtypeStruct((NP, D), jnp.float32),
            jax.ShapeDtypeStruct((NP, D), jnp.float32),
        ],
    )(x2, emb_pad, wm0)


def _edge_prep_body(e_ref, wf_ref, b2_ref, ec_ref):
    pid = pl.program_id(0)
    eb = e_ref[0, :].reshape(BE, 1)
    lane = lax.broadcasted_iota(jnp.int32, (BE, D), 1)
    centers = jnp.where(lane < RBF, lane.astype(jnp.float32) / (RBF - 1), 1e4)
    rbf = jnp.exp(-10.0 * (eb - centers) ** 2)
    row = pid * BE + lax.broadcasted_iota(jnp.int32, (BE, D), 0)
    valid = row < E
    for l in range(L):
        ec = jnp.dot(rbf, wf_ref[l], preferred_element_type=jnp.float32)
        ec = ec + b2_ref[l]
        ec_ref[l] = jnp.where(valid, ec, -1e9)


def _edge_prep(e2, wf, b2):
    return pl.pallas_call(
        _edge_prep_body,
        grid=(EP // BE,),
        in_specs=[
            pl.BlockSpec((1, BE), lambda j: (j, 0)),
            pl.BlockSpec((L, D, D), lambda j: (0, 0, 0)),
            pl.BlockSpec((L, 1, D), lambda j: (0, 0, 0)),
        ],
        out_specs=pl.BlockSpec((L, BE, D), lambda j: (0, j, 0)),
        out_shape=jax.ShapeDtypeStruct((L, EP, D), jnp.float32),
    )(e2, wf, b2)


def _update_body(agg_ref, h_ref, wu_ref, bu_ref, wnext_ref, h2_ref, hw_ref):
    aggs = agg_ref[0] + agg_ref[1]
    upd = jnp.dot(aggs, wu_ref[...], preferred_element_type=jnp.float32)
    upd = jnp.maximum(upd + bu_ref[...], 0.0)
    h2 = upd + h_ref[...]
    h2_ref[...] = h2
    hw_ref[...] = jnp.dot(h2, wnext_ref[...], preferred_element_type=jnp.float32)


def _update_last_body(agg_ref, h_ref, wu_ref, bu_ref, h2_ref):
    aggs = agg_ref[0] + agg_ref[1]
    upd = jnp.dot(aggs, wu_ref[...], preferred_element_type=jnp.float32)
    upd = jnp.maximum(upd + bu_ref[...], 0.0)
    h2_ref[...] = upd + h_ref[...]


def _update(agg2, h, wu, bu, wnext):
    if wnext is not None:
        return pl.pallas_call(
            _update_body,
            grid=(NP // BN,),
            in_specs=[
                pl.BlockSpec((2, BN, D), lambda j: (0, j, 0)),
                pl.BlockSpec((BN, D), lambda j: (j, 0)),
                pl.BlockSpec((D, D), lambda j: (0, 0)),
                pl.BlockSpec((1, D), lambda j: (0, 0)),
                pl.BlockSpec((D, D), lambda j: (0, 0)),
            ],
            out_specs=[
                pl.BlockSpec((BN, D), lambda j: (j, 0)),
                pl.BlockSpec((BN, D), lambda j: (j, 0)),
            ],
            out_shape=[
                jax.ShapeDtypeStruct((NP, D), jnp.float32),
                jax.ShapeDtypeStruct((NP, D), jnp.float32),
            ],
        )(agg2, h, wu, bu, wnext)
    return pl.pallas_call(
        _update_last_body,
        grid=(NP // BN,),
        in_specs=[
            pl.BlockSpec((2, BN, D), lambda j: (0, j, 0)),
            pl.BlockSpec((BN, D), lambda j: (j, 0)),
            pl.BlockSpec((D, D), lambda j: (0, 0)),
            pl.BlockSpec((1, D), lambda j: (0, 0)),
        ],
        out_specs=pl.BlockSpec((BN, D), lambda j: (j, 0)),
        out_shape=jax.ShapeDtypeStruct((NP, D), jnp.float32),
    )(agg2, h, wu, bu)


def _head_body(i_ref, h_ref, wn_ref, bn_ref, out_ref, sums_ref, counts_ref):
    pid = pl.program_id(0)
    nsteps = pl.num_programs(0)

    @pl.when(pid == 0)
    def _init():
        sums_ref[...] = jnp.zeros((NG, D), jnp.float32)
        counts_ref[...] = jnp.zeros((NG, D), jnp.float32)

    seg = lax.broadcasted_iota(jnp.int32, (NG, BN), 0)
    ib = i_ref[0, :].reshape(1, BN)
    oh_t = (seg == ib).astype(jnp.float32)
    sums_ref[...] += jnp.dot(oh_t, h_ref[...], preferred_element_type=jnp.float32)
    counts_ref[...] += jnp.dot(oh_t, jnp.ones((BN, D), jnp.float32),
                               preferred_element_type=jnp.float32)

    @pl.when(pid == nsteps - 1)
    def _final():
        counts = counts_ref[...]
        mean = sums_ref[...] / jnp.maximum(counts, 1.0)
        pooled = jnp.dot(mean, wn_ref[...], preferred_element_type=jnp.float32)
        pooled = pooled + bn_ref[...]
        out_ref[...] = jnp.where(counts > 0.0, pooled, 0.0)


def _head(i2, h, wn_pad, bn_pad):
    return pl.pallas_call(
        _head_body,
        grid=(NP // BN,),
        in_specs=[
            pl.BlockSpec((1, BN), lambda j: (j, 0)),
            pl.BlockSpec((BN, D), lambda j: (j, 0)),
            pl.BlockSpec((D, D), lambda j: (0, 0)),
            pl.BlockSpec((1, D), lambda j: (0, 0)),
        ],
        out_specs=pl.BlockSpec((NG, D), lambda j: (0, 0)),
        out_shape=jax.ShapeDtypeStruct((NG, D), jnp.float32),
        scratch_shapes=[
            pltpu.VMEM((NG, D), jnp.float32),
            pltpu.VMEM((NG, D), jnp.float32),
        ],
    )(i2, h, wn_pad, bn_pad)


def _make_sweep(l_idx):
    @functools.partial(
        pl.kernel,
        out_type=jax.ShapeDtypeStruct((2, NP, D), jnp.float32),
        mesh=_mesh,
        scratch_types=[
            pltpu.VMEM((NCHUNK, CH), jnp.int32),
            pltpu.VMEM((NCHUNK, CH), jnp.int32),
            pltpu.VMEM((CH, D), jnp.float32),
            pltpu.VMEM((CH, D), jnp.float32),
            pltpu.VMEM_SHARED((NP, D), jnp.float32),
            pltpu.SemaphoreType.DMA,
            pltpu.SemaphoreType.DMA,
        ],
    )
    def sweep(hw_hbm, ec_hbm, src_hbm, dst_hbm, out_hbm,
              src_v, dst_v, g_v, c_v, agg_sh, semg, semc):
        c = lax.axis_index("c")
        s = lax.axis_index("s")
        wid = c * 16 + s
        base_row = wid * NCHUNK
        base_e = wid * (NCHUNK * CH)

        def zrow(r, carry):
            for k in range(D // 16):
                g_v[r, pl.ds(k * 16, 16)] = jnp.zeros((16,), jnp.float32)
            return carry
        lax.fori_loop(0, CH, zrow, 0)
        for t in range(ROWS_PER_SUB // CH):
            pltpu.sync_copy(g_v, agg_sh.at[pl.ds(s * ROWS_PER_SUB + t * CH, CH)])
        plsc.subcore_barrier()

        pltpu.sync_copy(src_hbm.at[pl.ds(base_row, NCHUNK)], src_v)
        pltpu.sync_copy(dst_hbm.at[pl.ds(base_row, NCHUNK)], dst_v)

        def chunk(j, carry):
            gd = pltpu.async_copy(hw_hbm.at[src_v.at[j]], g_v, semg)
            cd = pltpu.async_copy(
                ec_hbm.at[l_idx, pl.ds(base_e + j * CH, CH)], c_v, semc)
            gd.wait()
            cd.wait()

            def row(r, carry2):
                for k in range(D // 16):
                    sl = (r, pl.ds(k * 16, 16))
                    g_v[sl] = jnp.maximum(g_v[sl] + c_v[sl], 0.0)
                return carry2
            lax.fori_loop(0, CH, row, 0)
            pltpu.sync_copy(g_v, agg_sh.at[dst_v.at[j]], add=True)
            return carry
        lax.fori_loop(0, NCHUNK, chunk, 0)

        plsc.subcore_barrier()
        pltpu.sync_copy(
            agg_sh.at[pl.ds(s * ROWS_PER_SUB, ROWS_PER_SUB)],
            out_hbm.at[c, pl.ds(s * ROWS_PER_SUB, ROWS_PER_SUB)])

    return sweep


_sweeps = [_make_sweep(l) for l in range(L)]


def _sweep_sc(hw, ec, src2, dst2, l_idx):
    return _sweeps[l_idx](hw, ec, src2, dst2)


def kernel(x, a, e, i, emb, W_e, b_e, W_msg, b_msg, W_upd, b_upd, W_n, b_n):
    f32 = jnp.float32
    xq = jnp.squeeze(x, axis=1).astype(jnp.int32)
    x2 = jnp.concatenate([xq, jnp.zeros((NP - N,), jnp.int32)]).reshape(NP // BN, BN)

    pad_idx = (jnp.arange(EP - E, dtype=jnp.int32) % N)
    src2 = jnp.concatenate([a[0].astype(jnp.int32), pad_idx]).reshape(EP // CH, CH)
    dst2 = jnp.concatenate([a[1].astype(jnp.int32), pad_idx]).reshape(EP // CH, CH)

    eq = jnp.squeeze(e, axis=1).astype(f32)
    e2 = jnp.concatenate([eq, jnp.zeros((EP - E,), f32)]).reshape(EP // BE, BE)

    i2 = jnp.concatenate([i.astype(jnp.int32),
                          jnp.full((NP - N,), NG, jnp.int32)]).reshape(NP // BN, BN)

    emb_pad = jnp.zeros((D, D), f32).at[:NUM_ELEM].set(emb.astype(f32))

    W_msg = W_msg.astype(f32)
    wm_h = W_msg[:, :D, :]
    wm_e = W_msg[:, D:, :]
    wf = jnp.zeros((L, D, D), f32).at[:, :RBF, :].set(
        jnp.einsum("kd,ldo->lko", W_e.astype(f32), wm_e))
    b2 = (jnp.einsum("d,ldo->lo", b_e.astype(f32), wm_e)
          + b_msg.astype(f32)).reshape(L, 1, D)

    wn_pad = jnp.zeros((D, D), f32).at[:, :12].set(W_n.astype(f32))
    bn_pad = jnp.zeros((1, D), f32).at[0, :12].set(b_n.astype(f32))

    ec = _edge_prep(e2, wf, b2)
    h, hw = _embed(x2, emb_pad, wm_h[0])
    for l in range(L):
        agg2 = _sweep_sc(hw, ec, src2, dst2, l)
        wnext = wm_h[l + 1] if l + 1 < L else None
        bu = b_upd[l].astype(f32).reshape(1, D)
        if wnext is not None:
            h, hw = _update(agg2, h, W_upd[l].astype(f32), bu, wnext)
        else:
            h = _update(agg2, h, W_upd[l].astype(f32), bu, None)

    pooled = _head(i2, h, wn_pad, bn_pad)
    return pooled[:, :12].reshape(NG, 3, 4)


# speedup vs baseline: 4.2846x; 4.2846x over previous
import functools

import jax
import jax.numpy as jnp
from jax import lax
from jax.experimental import pallas as pl
from jax.experimental.pallas import tpu as pltpu
from jax.experimental.pallas import tpu_sc as plsc

N, E, D, RBF, NG, L, NUM_ELEM = 10000, 320000, 128, 10, 64, 4, 84
NP = 10240
EP = 327680
CH = 128
NCHUNK = 80
BN = 1024
BE = 2048
ROWS_PER_SUB = NP // 16


def _embed_body(x_ref, emb_ref, wm0_ref, h_ref, hw_ref):
    xb = x_ref[0, 0, :].reshape(BN, 1)
    lane = lax.broadcasted_iota(jnp.int32, (BN, D), 1)
    oh = (lane == xb).astype(jnp.float32)
    h = jnp.dot(oh, emb_ref[...], preferred_element_type=jnp.float32)
    h_ref[...] = h
    hw_ref[...] = jnp.dot(h, wm0_ref[...], preferred_element_type=jnp.float32)


def _embed(x2, emb_pad, wm0):
    return pl.pallas_call(
        _embed_body,
        grid=(NP // BN,),
        in_specs=[
            pl.BlockSpec((1, 1, BN), lambda j: (j, 0, 0)),
            pl.BlockSpec((D, D), lambda j: (0, 0)),
            pl.BlockSpec((D, D), lambda j: (0, 0)),
        ],
        out_specs=[
            pl.BlockSpec((BN, D), lambda j: (j, 0)),
            pl.BlockSpec((BN, D), lambda j: (j, 0)),
        ],
        out_shape=[
            jax.ShapeDtypeStruct((NP, D), jnp.float32),
            jax.ShapeDtypeStruct((NP, D), jnp.float32),
        ],
    )(x2, emb_pad, wm0)


def _edge_prep_body(e_ref, wf_ref, b2_ref, ec_ref):
    pid = pl.program_id(0)
    eb = e_ref[0, 0, :].reshape(BE, 1)
    lane = lax.broadcasted_iota(jnp.int32, (BE, D), 1)
    centers = jnp.where(lane < RBF, lane.astype(jnp.float32) / (RBF - 1), 1e4)
    rbf = jnp.exp(-10.0 * (eb - centers) ** 2)
    row = pid * BE + lax.broadcasted_iota(jnp.int32, (BE, D), 0)
    valid = row < E
    for l in range(L):
        ec = jnp.dot(rbf, wf_ref[l], preferred_element_type=jnp.float32)
        ec = ec + b2_ref[l]
        ec_ref[l] = jnp.where(valid, ec, -1e9)


def _edge_prep(e2, wf, b2):
    return pl.pallas_call(
        _edge_prep_body,
        grid=(EP // BE,),
        in_specs=[
            pl.BlockSpec((1, 1, BE), lambda j: (j, 0, 0)),
            pl.BlockSpec((L, D, D), lambda j: (0, 0, 0)),
            pl.BlockSpec((L, 1, D), lambda j: (0, 0, 0)),
        ],
        out_specs=pl.BlockSpec((L, BE, D), lambda j: (0, j, 0)),
        out_shape=jax.ShapeDtypeStruct((L, EP, D), jnp.float32),
    )(e2, wf, b2)


def _agg_sum(agg_ref):
    return agg_ref[0] + agg_ref[1]


def _update_body(agg_ref, h_ref, wu_ref, bu_ref, wnext_ref, h2_ref, hw_ref):
    upd = jnp.dot(_agg_sum(agg_ref), wu_ref[...],
                  preferred_element_type=jnp.float32)
    upd = jnp.maximum(upd + bu_ref[...], 0.0)
    h2 = upd + h_ref[...]
    h2_ref[...] = h2
    hw_ref[...] = jnp.dot(h2, wnext_ref[...], preferred_element_type=jnp.float32)


def _update_last_body(agg_ref, h_ref, wu_ref, bu_ref, h2_ref):
    upd = jnp.dot(_agg_sum(agg_ref), wu_ref[...],
                  preferred_element_type=jnp.float32)
    upd = jnp.maximum(upd + bu_ref[...], 0.0)
    h2_ref[...] = upd + h_ref[...]


def _update(agg2, h, wu, bu, wnext):
    if wnext is not None:
        return pl.pallas_call(
            _update_body,
            grid=(NP // BN,),
            in_specs=[
                pl.BlockSpec((2, BN, D), lambda j: (0, j, 0)),
                pl.BlockSpec((BN, D), lambda j: (j, 0)),
                pl.BlockSpec((D, D), lambda j: (0, 0)),
                pl.BlockSpec((1, D), lambda j: (0, 0)),
                pl.BlockSpec((D, D), lambda j: (0, 0)),
            ],
            out_specs=[
                pl.BlockSpec((BN, D), lambda j: (j, 0)),
                pl.BlockSpec((BN, D), lambda j: (j, 0)),
            ],
            out_shape=[
                jax.ShapeDtypeStruct((NP, D), jnp.float32),
                jax.ShapeDtypeStruct((NP, D), jnp.float32),
            ],
        )(agg2, h, wu, bu, wnext)
    return pl.pallas_call(
        _update_last_body,
        grid=(NP // BN,),
        in_specs=[
            pl.BlockSpec((2, BN, D), lambda j: (0, j, 0)),
            pl.BlockSpec((BN, D), lambda j: (j, 0)),
            pl.BlockSpec((D, D), lambda j: (0, 0)),
            pl.BlockSpec((1, D), lambda j: (0, 0)),
        ],
        out_specs=pl.BlockSpec((BN, D), lambda j: (j, 0)),
        out_shape=jax.ShapeDtypeStruct((NP, D), jnp.float32),
    )(agg2, h, wu, bu)


def _head_body(i_ref, h_ref, wn_ref, bn_ref, out_ref, sums_ref, counts_ref):
    pid = pl.program_id(0)
    nsteps = pl.num_programs(0)

    @pl.when(pid == 0)
    def _init():
        sums_ref[...] = jnp.zeros((NG, D), jnp.float32)
        counts_ref[...] = jnp.zeros((NG, D), jnp.float32)

    seg = lax.broadcasted_iota(jnp.int32, (NG, BN), 0)
    ib = i_ref[0, 0, :].reshape(1, BN)
    oh_t = (seg == ib).astype(jnp.float32)
    sums_ref[...] += jnp.dot(oh_t, h_ref[...], preferred_element_type=jnp.float32)
    counts_ref[...] += jnp.dot(oh_t, jnp.ones((BN, D), jnp.float32),
                               preferred_element_type=jnp.float32)

    @pl.when(pid == nsteps - 1)
    def _final():
        counts = counts_ref[...]
        mean = sums_ref[...] / jnp.maximum(counts, 1.0)
        pooled = jnp.dot(mean, wn_ref[...], preferred_element_type=jnp.float32)
        pooled = pooled + bn_ref[...]
        out_ref[...] = jnp.where(counts > 0.0, pooled, 0.0)


def _head(i2, h, wn_pad, bn_pad):
    return pl.pallas_call(
        _head_body,
        grid=(NP // BN,),
        in_specs=[
            pl.BlockSpec((1, 1, BN), lambda j: (j, 0, 0)),
            pl.BlockSpec((BN, D), lambda j: (j, 0)),
            pl.BlockSpec((D, D), lambda j: (0, 0)),
            pl.BlockSpec((1, D), lambda j: (0, 0)),
        ],
        out_specs=pl.BlockSpec((NG, D), lambda j: (0, 0)),
        out_shape=jax.ShapeDtypeStruct((NG, D), jnp.float32),
        scratch_shapes=[
            pltpu.VMEM((NG, D), jnp.float32),
            pltpu.VMEM((NG, D), jnp.float32),
        ],
    )(i2, h, wn_pad, bn_pad)


NIDX = NCHUNK // 2


@functools.cache
def _make_sweep(l_idx):
    mesh = plsc.VectorSubcoreMesh(core_axis_name="c", subcore_axis_name="s")

    @functools.partial(
        pl.kernel,
        out_type=jax.ShapeDtypeStruct((2 * NP, D), jnp.float32),
        mesh=mesh,
        scratch_types=[
            pltpu.VMEM((NIDX, CH), jnp.int32),
            pltpu.VMEM((NIDX, CH), jnp.int32),
            pltpu.VMEM((CH, D), jnp.float32),
            pltpu.VMEM((CH, D), jnp.float32),
            pltpu.VMEM_SHARED((NP, D), jnp.float32),
            pltpu.SemaphoreType.DMA,
            pltpu.SemaphoreType.DMA,
        ],
    )
    def sweep(hw_hbm, ec_hbm, src_hbm, dst_hbm, zero_hbm, out_hbm,
              src_v, dst_v, g_v, c_v, agg_sh, semg, semc):
        c = lax.axis_index("c")
        s = lax.axis_index("s")
        wid = c * 16 + s
        base_row = wid * NCHUNK
        base_e = l_idx * EP + wid * (NCHUNK * CH)

        for t in range(16):
            @pl.when(s == t)
            def _zero(t=t):
                pltpu.sync_copy(
                    zero_hbm.at[pl.ds(t * ROWS_PER_SUB, ROWS_PER_SUB)],
                    agg_sh.at[pl.ds(t * ROWS_PER_SUB, ROWS_PER_SUB)])
        plsc.subcore_barrier()

        for half in range(2):
            idx_row0 = base_row + half * NIDX
            pltpu.sync_copy(src_hbm.at[pl.ds(idx_row0, NIDX)], src_v)
            pltpu.sync_copy(dst_hbm.at[pl.ds(idx_row0, NIDX)], dst_v)
            ec_row0 = base_e + half * (NIDX * CH)

            def chunk(j, carry):
                gd = pltpu.async_copy(hw_hbm.at[src_v.at[j]], g_v, semg)
                cd = pltpu.async_copy(
                    ec_hbm.at[pl.ds(ec_row0 + j * CH, CH)], c_v, semc)
                gd.wait()
                cd.wait()

                def row(r, carry2):
                    for k in range(D // 16):
                        sl = (r, pl.ds(k * 16, 16))
                        g_v[sl] = jnp.maximum(g_v[sl] + c_v[sl], 0.0)
                    return carry2
                lax.fori_loop(0, CH, row, 0)
                pltpu.sync_copy(g_v, agg_sh.at[dst_v.at[j]], add=True)
                return carry
            lax.fori_loop(0, NIDX, chunk, 0)

        plsc.subcore_barrier()
        for t in range(16):
            @pl.when(s == t)
            def _writeout(t=t):
                pltpu.sync_copy(
                    agg_sh.at[pl.ds(t * ROWS_PER_SUB, ROWS_PER_SUB)],
                    out_hbm.at[pl.ds(c * NP + t * ROWS_PER_SUB, ROWS_PER_SUB)])

    return sweep


def _sweep_sc(hw, ec, src2, dst2, zeros, l_idx):
    flat = _make_sweep(l_idx)(hw, ec.reshape(L * EP, D), src2, dst2, zeros)
    return flat.reshape(2, NP, D)


def kernel(x, a, e, i, emb, W_e, b_e, W_msg, b_msg, W_upd, b_upd, W_n, b_n):
    f32 = jnp.float32
    xq = jnp.squeeze(x, axis=1).astype(jnp.int32)
    x2 = jnp.concatenate([xq, jnp.zeros((NP - N,), jnp.int32)]).reshape(NP // BN, 1, BN)

    pad_idx = (jnp.arange(EP - E, dtype=jnp.int32) % N)
    src2 = jnp.concatenate([a[0].astype(jnp.int32), pad_idx]).reshape(EP // CH, CH)
    dst2 = jnp.concatenate([a[1].astype(jnp.int32), pad_idx]).reshape(EP // CH, CH)

    eq = jnp.squeeze(e, axis=1).astype(f32)
    e2 = jnp.concatenate([eq, jnp.zeros((EP - E,), f32)]).reshape(EP // BE, 1, BE)

    i2 = jnp.concatenate([i.astype(jnp.int32),
                          jnp.full((NP - N,), NG, jnp.int32)]).reshape(NP // BN, 1, BN)

    emb_pad = jnp.zeros((D, D), f32).at[:NUM_ELEM].set(emb.astype(f32))

    W_msg = W_msg.astype(f32)
    wm_h = W_msg[:, :D, :]
    wm_e = W_msg[:, D:, :]
    wf = jnp.zeros((L, D, D), f32).at[:, :RBF, :].set(
        jnp.einsum("kd,ldo->lko", W_e.astype(f32), wm_e))
    b2 = (jnp.einsum("d,ldo->lo", b_e.astype(f32), wm_e)
          + b_msg.astype(f32)).reshape(L, 1, D)

    wn_pad = jnp.zeros((D, D), f32).at[:, :12].set(W_n.astype(f32))
    bn_pad = jnp.zeros((1, D), f32).at[0, :12].set(b_n.astype(f32))

    ec = _edge_prep(e2, wf, b2)
    h, hw = _embed(x2, emb_pad, wm_h[0])
    zeros = jnp.zeros((NP, D), f32)
    for l in range(L):
        agg2 = _sweep_sc(hw, ec, src2, dst2, zeros, l)
        wnext = wm_h[l + 1] if l + 1 < L else None
        bu = b_upd[l].astype(f32).reshape(1, D)
        if wnext is not None:
            h, hw = _update(agg2, h, W_upd[l].astype(f32), bu, wnext)
        else:
            h = _update(agg2, h, W_upd[l].astype(f32), bu, None)

    pooled = _head(i2, h, wn_pad, bn_pad)
    return pooled[:, :12].reshape(NG, 3, 4)
